# Initial kernel scaffold; baseline (speedup 1.0000x reference)
import jax
import jax.numpy as jnp
from jax.experimental import pallas as pl


def kernel(x, index, weight, bias, decay_value, cache):
    raise NotImplementedError("write your pallas kernel here")


# speedup vs baseline: 3.0683x; 3.0683x over previous
import jax
import jax.numpy as jnp
from jax.experimental import pallas as pl
from jax.experimental.pallas import tpu as pltpu

HEADS = 2048
HEAD_DIM = 2048
H2 = HEADS // 2
BD = 512
BH = 512
LANES = 128


def _body(idx_ref, x_ref, w_ref, b_ref, dv_ref, c_ref, out_ref, nc_ref):
    j = pl.program_id(0)
    col = idx_ref[0] % LANES
    lane = jax.lax.broadcasted_iota(jnp.int32, (BH, LANES), 1)
    sel = lane == col
    zero = jnp.float32(0.0)
    wv = jnp.sum(jnp.where(sel, w_ref[...], zero), axis=1, keepdims=True)
    bv = jnp.sum(jnp.where(sel, b_ref[...], zero), axis=1, keepdims=True)
    dv1 = jnp.clip(dv_ref[1, 0], 0.9, 1.0)
    is_row = (j * BH) >= H2
    one = jnp.float32(1.0)
    a = jnp.where(is_row, wv, one)
    m = jnp.where(is_row, one, wv)
    xb = x_ref[...]
    cb = c_ref[...]
    t = a * xb.T + dv1 * cb
    nc_ref[...] = t
    out_ref[...] = (m * t + bv).T


def kernel(x, index, weight, bias, decay_value, cache):
    idx = jnp.asarray(index, dtype=jnp.int32).reshape(1)
    grid = (HEADS // BH, HEAD_DIM // BD)
    grid_spec = pltpu.PrefetchScalarGridSpec(
        num_scalar_prefetch=1,
        grid=grid,
        in_specs=[
            pl.BlockSpec((BD, BH), lambda j, i, s: (i, j)),
            pl.BlockSpec((BH, LANES), lambda j, i, s: (j, s[0] // LANES)),
            pl.BlockSpec((BH, LANES), lambda j, i, s: (j, s[0] // LANES)),
            pl.BlockSpec(memory_space=pltpu.SMEM),
            pl.BlockSpec((BH, BD), lambda j, i, s: (j, i)),
        ],
        out_specs=[
            pl.BlockSpec((BD, BH), lambda j, i, s: (i, j)),
            pl.BlockSpec((BH, BD), lambda j, i, s: (j, i)),
        ],
    )
    out, nc = pl.pallas_call(
        _body,
        grid_spec=grid_spec,
        out_shape=[
            jax.ShapeDtypeStruct((HEAD_DIM, HEADS), jnp.float32),
            jax.ShapeDtypeStruct((HEADS, HEAD_DIM), jnp.float32),
        ],
    )(idx, x, weight, bias, decay_value, cache)
    return out, nc


# speedup vs baseline: 3.0854x; 1.0056x over previous
import jax
import jax.numpy as jnp
from jax.experimental import pallas as pl
from jax.experimental.pallas import tpu as pltpu

HEADS = 2048
HEAD_DIM = 2048
H2 = HEADS // 2
BD = 512
BH = 512
LANES = 128


def _body(idx_ref, x_ref, w_ref, b_ref, dv_ref, c_ref, out_ref, nc_ref):
    j = pl.program_id(0)
    col = idx_ref[0] % LANES
    lane = jax.lax.broadcasted_iota(jnp.int32, (BH, LANES), 1)
    sel = lane == col
    zero = jnp.float32(0.0)
    wv = jnp.sum(jnp.where(sel, w_ref[...], zero), axis=1, keepdims=True)
    bv = jnp.sum(jnp.where(sel, b_ref[...], zero), axis=1, keepdims=True)
    dv1 = jnp.clip(dv_ref[1, 0], 0.9, 1.0)
    is_row = (j * BH) >= H2
    one = jnp.float32(1.0)
    a = jnp.where(is_row, wv, one)
    m = jnp.where(is_row, one, wv)
    xb = x_ref[...]
    cb = c_ref[...]
    t = a * xb.T + dv1 * cb
    nc_ref[...] = t
    out_ref[...] = (m * t + bv).T


def kernel(x, index, weight, bias, decay_value, cache):
    idx = jnp.asarray(index, dtype=jnp.int32).reshape(1)
    grid = (HEADS // BH, HEAD_DIM // BD)
    grid_spec = pltpu.PrefetchScalarGridSpec(
        num_scalar_prefetch=1,
        grid=grid,
        in_specs=[
            pl.BlockSpec((BD, BH), lambda j, i, s: (i, j)),
            pl.BlockSpec((BH, LANES), lambda j, i, s: (j, s[0] // LANES)),
            pl.BlockSpec((BH, LANES), lambda j, i, s: (j, s[0] // LANES)),
            pl.BlockSpec(memory_space=pltpu.SMEM),
            pl.BlockSpec((BH, BD), lambda j, i, s: (j, i)),
        ],
        out_specs=[
            pl.BlockSpec((BD, BH), lambda j, i, s: (i, j)),
            pl.BlockSpec((BH, BD), lambda j, i, s: (j, i)),
        ],
    )
    out, nc = pl.pallas_call(
        _body,
        grid_spec=grid_spec,
        compiler_params=pltpu.CompilerParams(
            dimension_semantics=("parallel", "parallel")),
        out_shape=[
            jax.ShapeDtypeStruct((HEAD_DIM, HEADS), jnp.float32),
            jax.ShapeDtypeStruct((HEADS, HEAD_DIM), jnp.float32),
        ],
    )(idx, x, weight, bias, decay_value, cache)
    return out, nc


# speedup vs baseline: 3.7790x; 1.2248x over previous
import jax
import jax.numpy as jnp
from jax.experimental import pallas as pl
from jax.experimental.pallas import tpu as pltpu

HEADS = 2048
HEAD_DIM = 2048
H2 = HEADS // 2
BD = 1024
BH = 1024
LANES = 128


def _body(idx_ref, x_ref, w_ref, b_ref, dv_ref, c_ref, out_ref, nc_ref):
    j = pl.program_id(0)
    col = idx_ref[0] % LANES
    lane = jax.lax.broadcasted_iota(jnp.int32, (BH, LANES), 1)
    sel = lane == col
    zero = jnp.float32(0.0)
    wv = jnp.sum(jnp.where(sel, w_ref[...], zero), axis=1, keepdims=True)
    bv = jnp.sum(jnp.where(sel, b_ref[...], zero), axis=1, keepdims=True)
    dv1 = jnp.clip(dv_ref[1, 0], 0.9, 1.0)
    is_row = (j * BH) >= H2
    one = jnp.float32(1.0)
    a = jnp.where(is_row, wv, one)
    m = jnp.where(is_row, one, wv)
    xb = x_ref[...]
    cb = c_ref[...]
    t = a * xb.T + dv1 * cb
    nc_ref[...] = t
    out_ref[...] = (m * t + bv).T


def kernel(x, index, weight, bias, decay_value, cache):
    idx = jnp.asarray(index, dtype=jnp.int32).reshape(1)
    grid = (HEADS // BH, HEAD_DIM // BD)
    grid_spec = pltpu.PrefetchScalarGridSpec(
        num_scalar_prefetch=1,
        grid=grid,
        in_specs=[
            pl.BlockSpec((BD, BH), lambda j, i, s: (i, j)),
            pl.BlockSpec((BH, LANES), lambda j, i, s: (j, s[0] // LANES)),
            pl.BlockSpec((BH, LANES), lambda j, i, s: (j, s[0] // LANES)),
            pl.BlockSpec(memory_space=pltpu.SMEM),
            pl.BlockSpec((BH, BD), lambda j, i, s: (j, i)),
        ],
        out_specs=[
            pl.BlockSpec((BD, BH), lambda j, i, s: (i, j)),
            pl.BlockSpec((BH, BD), lambda j, i, s: (j, i)),
        ],
    )
    out, nc = pl.pallas_call(
        _body,
        grid_spec=grid_spec,
        compiler_params=pltpu.CompilerParams(
            dimension_semantics=("parallel", "parallel")),
        out_shape=[
            jax.ShapeDtypeStruct((HEAD_DIM, HEADS), jnp.float32),
            jax.ShapeDtypeStruct((HEADS, HEAD_DIM), jnp.float32),
        ],
    )(idx, x, weight, bias, decay_value, cache)
    return out, nc
